# Initial kernel scaffold; baseline (speedup 1.0000x reference)
import jax
import jax.numpy as jnp
from jax import lax
from jax.experimental import pallas as pl
from jax.experimental.pallas import tpu as pltpu
from jax.experimental.pallas import tpu_sc as plsc

_INPUT_SCALE = 0.0627
_QMAX = 255.0

_ROWS = 128
_COLS = 32768
_LANES = 16
_NC = 2
_NS = 16
_NW = _NC * _NS
_ROWS_PER_W = _ROWS // _NW
_VECS = _COLS // _LANES


def _body(in_hbm, lut_hbm, out_hbm, xbuf, lut):
    wid = lax.axis_index("s") * _NC + lax.axis_index("c")
    pltpu.sync_copy(lut_hbm, lut)
    for rr in range(_ROWS_PER_W):
        row = wid * _ROWS_PER_W + rr
        pltpu.sync_copy(in_hbm.at[row], xbuf)

        def p1(i, acc):
            x = xbuf[pl.ds(i * _LANES, _LANES)]
            q = jnp.clip(x / _INPUT_SCALE, 0.0, _QMAX).astype(jnp.int32)
            e = plsc.load_gather(lut, [q])
            xbuf[pl.ds(i * _LANES, _LANES)] = e
            return acc + e

        acc = lax.fori_loop(0, _VECS, p1, jnp.zeros((_LANES,), jnp.float32))
        inv = 1.0 / jnp.sum(acc)

        def p2(i, c):
            xbuf[pl.ds(i * _LANES, _LANES)] = xbuf[pl.ds(i * _LANES, _LANES)] * inv
            return c

        lax.fori_loop(0, _VECS, p2, 0)
        pltpu.sync_copy(xbuf, out_hbm.at[row])


def kernel(input, array):
    mesh = plsc.VectorSubcoreMesh(core_axis_name="c", subcore_axis_name="s")
    f = pl.kernel(
        _body,
        mesh=mesh,
        out_type=jax.ShapeDtypeStruct((_ROWS, _COLS), jnp.float32),
        scratch_types=[
            pltpu.VMEM((_COLS,), jnp.float32),
            pltpu.VMEM((256,), jnp.float32),
        ],
    )
    return f(input, array)


# speedup vs baseline: 219.9333x; 219.9333x over previous
import jax
import jax.numpy as jnp
from jax import lax
from jax.experimental import pallas as pl
from jax.experimental.pallas import tpu as pltpu
from jax.experimental.pallas import tpu_sc as plsc

_INPUT_SCALE = 0.0627
_QMAX = 255.0

_ROWS = 128
_COLS = 32768
_LANES = 16
_NC = 2
_NS = 16
_NW = _NC * _NS
_ROWS_PER_W = _ROWS // _NW
_VECS = _COLS // _LANES


def _body(in_hbm, lut_hbm, out_hbm, xbuf, lut):
    wid = lax.axis_index("s") * _NC + lax.axis_index("c")
    pltpu.sync_copy(lut_hbm, lut)
    for rr in range(_ROWS_PER_W):
        row = wid * _ROWS_PER_W + rr
        pltpu.sync_copy(in_hbm.at[row], xbuf)

        def p1(i, acc):
            x = xbuf[pl.ds(i * _LANES, _LANES)]
            q = jnp.clip(x / _INPUT_SCALE, 0.0, _QMAX).astype(jnp.int32)
            e = plsc.load_gather(lut, [q])
            xbuf[pl.ds(i * _LANES, _LANES)] = e
            return acc + e

        acc = lax.fori_loop(0, _VECS, p1, jnp.zeros((_LANES,), jnp.float32))
        total = jnp.broadcast_to(jnp.sum(acc), (_LANES,))
        inv = jnp.ones((_LANES,), jnp.float32) / total

        def p2(i, c):
            xbuf[pl.ds(i * _LANES, _LANES)] = xbuf[pl.ds(i * _LANES, _LANES)] * inv
            return c

        lax.fori_loop(0, _VECS, p2, 0)
        pltpu.sync_copy(xbuf, out_hbm.at[row])


def kernel(input, array):
    mesh = plsc.VectorSubcoreMesh(core_axis_name="c", subcore_axis_name="s")
    f = pl.kernel(
        _body,
        mesh=mesh,
        out_type=jax.ShapeDtypeStruct((_ROWS, _COLS), jnp.float32),
        scratch_types=[
            pltpu.VMEM((_COLS,), jnp.float32),
            pltpu.VMEM((256,), jnp.float32),
        ],
        compiler_params=pltpu.CompilerParams(needs_layout_passes=False),
    )
    return f(input, array)


# speedup vs baseline: 941.7142x; 4.2818x over previous
import jax
import jax.numpy as jnp
from jax import lax
from jax.experimental import pallas as pl
from jax.experimental.pallas import tpu as pltpu
from jax.experimental.pallas import tpu_sc as plsc

_INPUT_SCALE = 0.0627
_QMAX = 255.0

_ROWS = 128
_COLS = 32768
_LANES = 16
_NC = 2
_NS = 16
_NW = _NC * _NS
_ROWS_PER_W = _ROWS // _NW
_NBUF = 3
_UNROLL = 8


def _body(in_hbm, lut_hbm, out_hbm,
          buf0, buf1, buf2, lut,
          i0, i1, i2, o0, o1, o2):
    wid = lax.axis_index("s") * _NC + lax.axis_index("c")
    base = wid * _ROWS_PER_W
    pltpu.sync_copy(lut_hbm, lut)

    bufs = (buf0, buf1, buf2)
    isems = (i0, i1, i2)
    osems = (o0, o1, o2)
    in_cp = [None] * _ROWS_PER_W
    out_cp = [None] * _ROWS_PER_W

    in_cp[0] = pltpu.async_copy(in_hbm.at[base], bufs[0], isems[0])
    in_cp[1] = pltpu.async_copy(in_hbm.at[base + 1], bufs[1], isems[1])

    for rr in range(_ROWS_PER_W):
        b = rr % _NBUF
        buf = bufs[b]
        in_cp[rr].wait()

        @plsc.parallel_loop(0, _COLS, _LANES, unroll=_UNROLL,
                            carry=jnp.zeros((_LANES,), jnp.float32))
        def p1(i, acc, buf=buf):
            x = buf[pl.ds(i, _LANES)]
            q = jnp.clip(x / _INPUT_SCALE, 0.0, _QMAX).astype(jnp.int32)
            e = plsc.load_gather(lut, [q])
            buf[pl.ds(i, _LANES)] = e
            return acc + e

        total = jnp.broadcast_to(jnp.sum(p1), (_LANES,))
        inv = jnp.ones((_LANES,), jnp.float32) / total

        @plsc.parallel_loop(0, _COLS, _LANES, unroll=_UNROLL)
        def p2(i, buf=buf, inv=inv):
            buf[pl.ds(i, _LANES)] = buf[pl.ds(i, _LANES)] * inv

        if rr >= 1:
            out_cp[rr - 1].wait()
        if rr + 2 < _ROWS_PER_W:
            nb = (rr + 2) % _NBUF
            in_cp[rr + 2] = pltpu.async_copy(
                in_hbm.at[base + rr + 2], bufs[nb], isems[nb])
        out_cp[rr] = pltpu.async_copy(buf, out_hbm.at[base + rr], osems[b])

    out_cp[_ROWS_PER_W - 1].wait()


def kernel(input, array):
    mesh = plsc.VectorSubcoreMesh(core_axis_name="c", subcore_axis_name="s")
    f = pl.kernel(
        _body,
        mesh=mesh,
        out_type=jax.ShapeDtypeStruct((_ROWS, _COLS), jnp.float32),
        scratch_types=[
            pltpu.VMEM((_COLS,), jnp.float32),
            pltpu.VMEM((_COLS,), jnp.float32),
            pltpu.VMEM((_COLS,), jnp.float32),
            pltpu.VMEM((256,), jnp.float32),
            pltpu.SemaphoreType.DMA,
            pltpu.SemaphoreType.DMA,
            pltpu.SemaphoreType.DMA,
            pltpu.SemaphoreType.DMA,
            pltpu.SemaphoreType.DMA,
            pltpu.SemaphoreType.DMA,
        ],
        compiler_params=pltpu.CompilerParams(needs_layout_passes=False),
    )
    return f(input, array)
